# baseline (speedup 1.0000x reference)
import functools

import jax
import jax.numpy as jnp
from jax import lax
from jax.experimental import pallas as pl
from jax.experimental.pallas import tpu as pltpu
from jax.experimental.pallas import tpu_sc as plsc

_NC = 2
_NS = 16
_NW = _NC * _NS

_BATCH = 8
_CHW = 128 * 128 * 128
_PER_W = _CHW // 2
_DEPTH = 2
_CHUNK = 32 * 1024
_NCHUNK = _PER_W // _CHUNK
_NSLOT = 2 * _DEPTH
_TOTAL = _BATCH * 2 * _CHW


def _copy_region(src_hbm, out_hbm, k, half_off, shared, base, tbufs,
                 lsems, ssems):
    src_off = k * _PER_W
    bb = k // 2
    hh = k % 2
    dst_off = bb * (2 * _CHW) + half_off + hh * _PER_W

    loads = [None] * _NCHUNK
    stores = [None] * _NCHUNK

    def buf_at(i):
        slot = (i // 2) % _DEPTH
        if i % 2 == 0:
            return shared.at[pl.ds(base + slot * _CHUNK, _CHUNK)]
        return tbufs[slot]

    def sem_at(i):
        return (i % 2) * _DEPTH + (i // 2) % _DEPTH

    def load(i):
        return pltpu.async_copy(
            src_hbm.at[pl.ds(src_off + i * _CHUNK, _CHUNK)],
            buf_at(i), lsems[sem_at(i)])

    def store(i):
        return pltpu.async_copy(
            buf_at(i),
            out_hbm.at[pl.ds(dst_off + i * _CHUNK, _CHUNK)],
            ssems[sem_at(i)])

    lookahead = _NSLOT - 1
    for i in range(min(lookahead, _NCHUNK)):
        loads[i] = load(i)
    for i in range(_NCHUNK):
        loads[i].wait()
        stores[i] = store(i)
        nxt = i + lookahead
        if nxt < _NCHUNK:
            if nxt - _NSLOT >= 0:
                stores[nxt - _NSLOT].wait()
            loads[nxt] = load(nxt)
    for i in range(max(0, _NCHUNK - _NSLOT), _NCHUNK):
        stores[i].wait()


def _sc_body(a_hbm, b_hbm, out_hbm, shared, *scratch):
    tbufs = scratch[:_DEPTH]
    lsems = scratch[_DEPTH:_DEPTH + _NSLOT]
    ssems = scratch[_DEPTH + _NSLOT:_DEPTH + 2 * _NSLOT]
    sid = lax.axis_index("s")
    wid = sid * _NC + lax.axis_index("c")
    base = sid * (_DEPTH * _CHUNK)

    @pl.when(wid < _NS)
    def _():
        _copy_region(a_hbm, out_hbm, wid, 0, shared, base, tbufs,
                     lsems, ssems)

    @pl.when(wid >= _NS)
    def _():
        _copy_region(b_hbm, out_hbm, wid - _NS, _CHW, shared, base, tbufs,
                     lsems, ssems)


_sc_concat = functools.partial(
    pl.kernel,
    mesh=plsc.VectorSubcoreMesh(core_axis_name="c", subcore_axis_name="s"),
    out_type=jax.ShapeDtypeStruct((_TOTAL,), jnp.float32),
    scratch_types=(
        [pltpu.VMEM_SHARED((_NS * _DEPTH * _CHUNK,), jnp.float32)]
        + [pltpu.VMEM((_CHUNK,), jnp.float32)] * _DEPTH
        + [pltpu.SemaphoreType.DMA] * (2 * _NSLOT)
    ),
)(_sc_body)


def kernel(a, b, scatter_a, scatter_b):
    B, C, H, W = a.shape
    out_flat = _sc_concat(a.reshape(-1), b.reshape(-1))
    return out_flat.reshape(B, 2 * C, H, W)


# speedup vs baseline: 1.0253x; 1.0253x over previous
import functools

import jax
import jax.numpy as jnp
from jax import lax
from jax.experimental import pallas as pl
from jax.experimental.pallas import tpu as pltpu
from jax.experimental.pallas import tpu_sc as plsc

_NC = 2
_NS = 16
_NW = _NC * _NS

_BATCH = 8
_CHW = 128 * 128 * 128
_PER_W = _CHW // 2
_DEPTH = 2
_CHUNK = 64 * 1024
_NCHUNK = _PER_W // _CHUNK
_TOTAL = _BATCH * 2 * _CHW


def _copy_region(src_hbm, out_hbm, k, half_off, shared, base, lsems, ssems):
    src_off = k * _PER_W
    bb = k // 2
    hh = k % 2
    dst_off = bb * (2 * _CHW) + half_off + hh * _PER_W

    loads = [None] * _NCHUNK
    stores = [None] * _NCHUNK

    def load(i):
        return pltpu.async_copy(
            src_hbm.at[pl.ds(src_off + i * _CHUNK, _CHUNK)],
            shared.at[pl.ds(base + (i % _DEPTH) * _CHUNK, _CHUNK)],
            lsems[i % _DEPTH])

    def store(i):
        return pltpu.async_copy(
            shared.at[pl.ds(base + (i % _DEPTH) * _CHUNK, _CHUNK)],
            out_hbm.at[pl.ds(dst_off + i * _CHUNK, _CHUNK)],
            ssems[i % _DEPTH])

    lookahead = _DEPTH - 1
    for i in range(lookahead):
        loads[i] = load(i)
    for i in range(_NCHUNK):
        loads[i].wait()
        stores[i] = store(i)
        nxt = i + lookahead
        if nxt < _NCHUNK:
            if nxt - _DEPTH >= 0:
                stores[nxt - _DEPTH].wait()
            loads[nxt] = load(nxt)
    for i in range(max(0, _NCHUNK - _DEPTH), _NCHUNK):
        stores[i].wait()


def _sc_body(a_hbm, b_hbm, out_hbm, shared, *scratch):
    lsems = scratch[:_DEPTH]
    ssems = scratch[_DEPTH:2 * _DEPTH]
    sid = lax.axis_index("s")
    wid = sid * _NC + lax.axis_index("c")
    base = sid * (_DEPTH * _CHUNK)

    @pl.when(wid < _NS)
    def _():
        _copy_region(a_hbm, out_hbm, wid, 0, shared, base, lsems, ssems)

    @pl.when(wid >= _NS)
    def _():
        _copy_region(b_hbm, out_hbm, wid - _NS, _CHW, shared, base, lsems, ssems)


_sc_concat = functools.partial(
    pl.kernel,
    mesh=plsc.VectorSubcoreMesh(core_axis_name="c", subcore_axis_name="s"),
    out_type=jax.ShapeDtypeStruct((_TOTAL,), jnp.float32),
    scratch_types=(
        [pltpu.VMEM_SHARED((_NS * _DEPTH * _CHUNK,), jnp.float32)]
        + [pltpu.SemaphoreType.DMA] * (2 * _DEPTH)
    ),
)(_sc_body)


def kernel(a, b, scatter_a, scatter_b):
    B, C, H, W = a.shape
    out_flat = _sc_concat(a.reshape(-1), b.reshape(-1))
    return out_flat.reshape(B, 2 * C, H, W)
